# baseline (speedup 1.0000x reference)
import functools

import jax
import jax.numpy as jnp
from jax import lax
from jax.experimental import pallas as pl
from jax.experimental.pallas import tpu as pltpu
from jax.experimental.pallas import tpu_sc as plsc

SKIP = 2
DELAY = 4
DIM = 16

NC = 2
NS = 16
NW = NC * NS

CH = 15632
SLAB = 2 * CH + 64
HP = 32
THL = 2 * CH + 64
THB = THL + 2 * HP


def _tde_body(n, n_pts, per_w, ts_hbm, y_hbm, slab_a, slab_b, obuf_a, obuf_b,
              thead, sem_sa, sem_sb, sem_oa, sem_ob):
    slab2 = [slab_a, slab_b]
    obuf2 = [obuf_a, obuf_b]
    sem_s = [sem_sa, sem_sb]
    sem_o = [sem_oa, sem_ob]
    wid = lax.axis_index("s") * NC + lax.axis_index("c")
    lanes2 = jnp.arange(16, dtype=jnp.int32) * 2
    lane = jnp.arange(16, dtype=jnp.int32)
    total = n_pts * DIM

    pltpu.sync_copy(ts_hbm.at[pl.ds(0, THL)], thead.at[pl.ds(HP, THL)])

    def chunk_params(gl):
        g = wid + NW * gl
        p0 = jnp.minimum(g * CH, total - CH)
        k0 = p0 // n_pts
        len1 = jnp.minimum((k0 + 1) * n_pts - p0, CH)
        a1 = p0 - (n_pts - 2) * k0
        sa = jnp.minimum(2 * a1, n - SLAB)
        delta = (2 * a1 - sa).astype(jnp.int32)
        return p0, k0, len1, delta, sa

    p0_n, k0_n, len1_n, delta_n, sa_n = chunk_params(0)
    slab_wait = [None, None]
    out_wait = [None, None]
    slab_wait[0] = pltpu.async_copy(
        ts_hbm.at[pl.ds(sa_n, SLAB)], slab2[0], sem_s[0]
    )

    for gl in range(per_w):
        pb = gl % 2
        p0, k0, len1, delta, sa = p0_n, k0_n, len1_n, delta_n, sa_n
        slab = slab2[pb]
        obuf = obuf2[pb]

        slab_wait[pb].wait()
        if gl + 1 < per_w:
            p0_n, k0_n, len1_n, delta_n, sa_n = chunk_params(gl + 1)
            slab_wait[1 - pb] = pltpu.async_copy(
                ts_hbm.at[pl.ds(sa_n, SLAB)], slab2[1 - pb], sem_s[1 - pb]
            )
        if out_wait[pb] is not None:
            out_wait[pb].wait()

        n1 = (len1 + 15) // 16

        @plsc.parallel_loop(0, n1, 1, unroll=8)
        def _(m):
            gi = delta + 32 * m + lanes2
            obuf[pl.ds(16 * m, 16)] = plsc.load_gather(
                slab, [jnp.minimum(gi, SLAB - 1)]
            )

        m2s = len1 // 16
        base2 = HP + 4 * (k0 + 1) - 2 * len1

        @plsc.parallel_loop(m2s + 1, CH // 16, 1, unroll=8)
        def _(m):
            gi = base2 + 32 * m + lanes2
            obuf[pl.ds(16 * m, 16)] = plsc.load_gather(
                thead, [jnp.minimum(gi, THB - 1)]
            )

        @pl.when(len1 < CH)
        def _():
            x = 16 * m2s + lane
            cur = obuf[pl.ds(16 * m2s, 16)]
            gi = base2 + 32 * m2s + lanes2
            alt = plsc.load_gather(thead, [jnp.minimum(gi, THB - 1)])
            obuf[pl.ds(16 * m2s, 16)] = jnp.where(x < len1, cur, alt)

        out_wait[pb] = pltpu.async_copy(
            obuf, y_hbm.at[pl.ds(p0, CH)], sem_o[pb]
        )

    out_wait[(per_w - 2) % 2].wait()
    out_wait[(per_w - 1) % 2].wait()


def kernel(timeSeries):
    n = timeSeries.shape[0]
    if n == 1:
        return timeSeries
    n_pts = (n - (DIM - 1) * DELAY) // SKIP
    total = n_pts * DIM
    n_chunks = (total + CH - 1) // CH
    per_w = (n_chunks + NW - 1) // NW

    mesh = plsc.VectorSubcoreMesh(
        core_axis_name="c", subcore_axis_name="s", num_cores=NC, num_subcores=NS
    )
    y_flat = pl.kernel(
        functools.partial(_tde_body, n, n_pts, per_w),
        out_type=jax.ShapeDtypeStruct((total,), jnp.float32),
        mesh=mesh,
        scratch_types=[
            pltpu.VMEM((SLAB,), jnp.float32),
            pltpu.VMEM((SLAB,), jnp.float32),
            pltpu.VMEM((CH,), jnp.float32),
            pltpu.VMEM((CH,), jnp.float32),
            pltpu.VMEM((THB,), jnp.float32),
            pltpu.SemaphoreType.DMA,
            pltpu.SemaphoreType.DMA,
            pltpu.SemaphoreType.DMA,
            pltpu.SemaphoreType.DMA,
        ],
        compiler_params=pltpu.CompilerParams(needs_layout_passes=False),
    )(timeSeries)
    return y_flat.reshape(DIM, n_pts).T


# speedup vs baseline: 7.0620x; 7.0620x over previous
import functools

import jax
import jax.numpy as jnp
from jax import lax
from jax.experimental import pallas as pl
from jax.experimental.pallas import tpu as pltpu
from jax.experimental.pallas import tpu_sc as plsc

SKIP = 2
DELAY = 4
DIM = 16

NC = 2
NS = 16
NW = NC * NS

CH = 15632
SLAB = 2 * CH + 64
HP = 32
THL = 2 * CH + 64
THB = THL + 2 * HP
BW = 32768


def _tde_body(n, pitch, per_w, ts_hbm, y_hbm, slab_a, slab_b, obuf_a, obuf_b,
              thead, sem_sa, sem_sb, sem_oa, sem_ob):
    slab2 = [slab_a, slab_b]
    obuf2 = [obuf_a, obuf_b]
    sem_s = [sem_sa, sem_sb]
    sem_o = [sem_oa, sem_ob]
    wid = lax.axis_index("s") * NC + lax.axis_index("c")
    lanes2 = jnp.arange(16, dtype=jnp.int32) * 2
    lane = jnp.arange(16, dtype=jnp.int32)
    total = pitch * DIM

    pltpu.sync_copy(ts_hbm.at[pl.ds(0, THL)], thead.at[pl.ds(HP, THL)])

    def chunk_params(gl):
        g = wid + NW * gl
        p0 = jnp.minimum(g * CH, total - CH)
        k0 = p0 // pitch
        len1 = jnp.minimum((k0 + 1) * pitch - p0, CH)
        a1 = p0 - (pitch - 2) * k0
        sa = jnp.minimum((2 * a1 // 8) * 8, n - SLAB)
        delta = (2 * a1 - sa).astype(jnp.int32)
        return p0, k0, len1, delta, sa

    p0_n, k0_n, len1_n, delta_n, sa_n = chunk_params(0)
    slab_wait = [None, None]
    out_wait = [None, None]
    slab_wait[0] = pltpu.async_copy(
        ts_hbm.at[pl.ds(sa_n, SLAB)], slab2[0], sem_s[0]
    )

    for gl in range(per_w):
        pb = gl % 2
        p0, k0, len1, delta, sa = p0_n, k0_n, len1_n, delta_n, sa_n
        slab = slab2[pb]
        obuf = obuf2[pb]

        slab_wait[pb].wait()
        if gl + 1 < per_w:
            p0_n, k0_n, len1_n, delta_n, sa_n = chunk_params(gl + 1)
            slab_wait[1 - pb] = pltpu.async_copy(
                ts_hbm.at[pl.ds(sa_n, SLAB)], slab2[1 - pb], sem_s[1 - pb]
            )
        if out_wait[pb] is not None:
            out_wait[pb].wait()

        n1 = (len1 + 15) // 16

        @plsc.parallel_loop(0, n1, 1, unroll=8)
        def _(m):
            gi = delta + 32 * m + lanes2
            obuf[pl.ds(16 * m, 16)] = plsc.load_gather(
                slab, [jnp.minimum(gi, SLAB - 1)]
            )

        m2s = len1 // 16
        base2 = HP + 4 * (k0 + 1) - 2 * len1

        @plsc.parallel_loop(m2s + 1, CH // 16, 1, unroll=8)
        def _(m):
            gi = base2 + 32 * m + lanes2
            obuf[pl.ds(16 * m, 16)] = plsc.load_gather(
                thead, [jnp.minimum(gi, THB - 1)]
            )

        @pl.when(len1 < CH)
        def _():
            x = 16 * m2s + lane
            cur = obuf[pl.ds(16 * m2s, 16)]
            gi = base2 + 32 * m2s + lanes2
            alt = plsc.load_gather(thead, [jnp.minimum(gi, THB - 1)])
            obuf[pl.ds(16 * m2s, 16)] = jnp.where(x < len1, cur, alt)

        out_wait[pb] = pltpu.async_copy(
            obuf, y_hbm.at[pl.ds(p0, CH)], sem_o[pb]
        )

    out_wait[(per_w - 2) % 2].wait()
    out_wait[(per_w - 1) % 2].wait()


def _retile_body(pitch, y_any, o_ref, sem):
    i = pl.program_id(0)
    j0 = i * BW
    descs = [
        pltpu.make_async_copy(
            y_any.at[pl.ds(k * pitch + j0, BW)], o_ref.at[k], sem
        )
        for k in range(DIM)
    ]
    for d in descs:
        d.start()
    for d in descs:
        d.wait()


def kernel(timeSeries):
    n = timeSeries.shape[0]
    if n == 1:
        return timeSeries
    n_pts = (n - (DIM - 1) * DELAY) // SKIP
    pitch = ((n_pts + 127) // 128) * 128
    total = pitch * DIM
    n_chunks = (total + CH - 1) // CH
    per_w = (n_chunks + NW - 1) // NW

    mesh = plsc.VectorSubcoreMesh(
        core_axis_name="c", subcore_axis_name="s", num_cores=NC, num_subcores=NS
    )
    y_flat = pl.kernel(
        functools.partial(_tde_body, n, pitch, per_w),
        out_type=jax.ShapeDtypeStruct((total + BW,), jnp.float32),
        mesh=mesh,
        scratch_types=[
            pltpu.VMEM((SLAB,), jnp.float32),
            pltpu.VMEM((SLAB,), jnp.float32),
            pltpu.VMEM((CH,), jnp.float32),
            pltpu.VMEM((CH,), jnp.float32),
            pltpu.VMEM((THB,), jnp.float32),
            pltpu.SemaphoreType.DMA,
            pltpu.SemaphoreType.DMA,
            pltpu.SemaphoreType.DMA,
            pltpu.SemaphoreType.DMA,
        ],
        compiler_params=pltpu.CompilerParams(needs_layout_passes=False),
    )(timeSeries)

    out_t = pl.pallas_call(
        functools.partial(_retile_body, pitch),
        out_shape=jax.ShapeDtypeStruct((DIM, n_pts), jnp.float32),
        grid=((n_pts + BW - 1) // BW,),
        in_specs=[pl.BlockSpec(memory_space=pl.ANY)],
        out_specs=pl.BlockSpec((DIM, BW), lambda i: (0, i)),
        scratch_shapes=[pltpu.SemaphoreType.DMA],
    )(y_flat)
    return out_t.T


# speedup vs baseline: 10.5819x; 1.4984x over previous
import functools

import jax
import jax.numpy as jnp
from jax import lax
from jax.experimental import pallas as pl
from jax.experimental.pallas import tpu as pltpu
from jax.experimental.pallas import tpu_sc as plsc

SKIP = 2
DELAY = 4
DIM = 16

NC = 2
NS = 16
NW = NC * NS

CH = 15632
SLAB = 2 * CH + 80
E4L = CH + 32
BW = 32768


def _tde_body(n, pitch, per_w, ts_hbm, y_hbm, slab, e0, e1, e2, e3,
              sem_s, sem_o):
    e4 = [e0, e1, e2, e3]
    wid = lax.axis_index("s") * NC + lax.axis_index("c")
    lanes2 = jnp.arange(16, dtype=jnp.int32) * 2

    for gl in range(per_w):
        band = wid + NW * gl
        j0 = jnp.minimum(band * CH, pitch - CH)
        sa = jnp.minimum(2 * j0, n - SLAB)
        delta = (2 * j0 - sa).astype(jnp.int32)

        pltpu.async_copy(ts_hbm.at[pl.ds(sa, SLAB)], slab, sem_s).wait()

        for c in range(4):
            ec = e4[c]

            @plsc.parallel_loop(0, E4L // 16, 1, unroll=8)
            def _(m, _c=c, _ec=ec):
                gi = delta + 4 * _c + 32 * m + lanes2
                _ec[pl.ds(16 * m, 16)] = plsc.load_gather(
                    slab, [jnp.minimum(gi, SLAB - 1)]
                )

        descs = [
            pltpu.make_async_copy(
                e4[k % 4].at[pl.ds(8 * (k // 4), CH)],
                y_hbm.at[pl.ds(k * pitch + j0, CH)],
                sem_o,
            )
            for k in range(DIM)
        ]
        for d in descs:
            d.start()
        for d in descs:
            d.wait()


def _retile_body(pitch, y_any, o_ref, sem):
    i = pl.program_id(0)
    j0 = i * BW
    descs = [
        pltpu.make_async_copy(
            y_any.at[pl.ds(k * pitch + j0, BW)], o_ref.at[k], sem
        )
        for k in range(DIM)
    ]
    for d in descs:
        d.start()
    for d in descs:
        d.wait()


def kernel(timeSeries):
    n = timeSeries.shape[0]
    if n == 1:
        return timeSeries
    n_pts = (n - (DIM - 1) * DELAY) // SKIP
    pitch = ((n_pts + 127) // 128) * 128
    total = pitch * DIM
    n_bands = (pitch + CH - 1) // CH
    per_w = (n_bands + NW - 1) // NW

    mesh = plsc.VectorSubcoreMesh(
        core_axis_name="c", subcore_axis_name="s", num_cores=NC, num_subcores=NS
    )
    y_flat = pl.kernel(
        functools.partial(_tde_body, n, pitch, per_w),
        out_type=jax.ShapeDtypeStruct((total + BW,), jnp.float32),
        mesh=mesh,
        scratch_types=[
            pltpu.VMEM((SLAB,), jnp.float32),
            pltpu.VMEM((E4L,), jnp.float32),
            pltpu.VMEM((E4L,), jnp.float32),
            pltpu.VMEM((E4L,), jnp.float32),
            pltpu.VMEM((E4L,), jnp.float32),
            pltpu.SemaphoreType.DMA,
            pltpu.SemaphoreType.DMA,
        ],
        compiler_params=pltpu.CompilerParams(needs_layout_passes=False),
    )(timeSeries)

    out_t = pl.pallas_call(
        functools.partial(_retile_body, pitch),
        out_shape=jax.ShapeDtypeStruct((DIM, n_pts), jnp.float32),
        grid=((n_pts + BW - 1) // BW,),
        in_specs=[pl.BlockSpec(memory_space=pl.ANY)],
        out_specs=pl.BlockSpec((DIM, BW), lambda i: (0, i)),
        scratch_shapes=[pltpu.SemaphoreType.DMA],
    )(y_flat)
    return out_t.T
